# baseline (device time: 35104 ns/iter reference)
import os as _os

import jax
import jax.numpy as jnp
from jax import lax
from jax.experimental import pallas as pl
from jax.experimental.pallas import tpu as pltpu

N_DEV = 32
N_PAIR = 16
R_HOPS = 8
L_HOPS = 7

_CY = [(0, 0), (1, 0), (2, 0), (3, 0), (3, 1), (3, 2), (3, 3), (2, 3),
       (2, 2), (2, 1), (1, 1), (1, 2), (1, 3), (0, 3), (0, 2), (0, 1)]
Y16 = [y for y, _ in _CY]
Z16 = [z for _, z in _CY]
P16 = [4 * z + y for y, z in _CY]
INV16 = [0] * N_PAIR
for _t, _p in enumerate(P16):
    INV16[_p] = _t


def _tab(idx, table):
    v = jnp.int32(table[0])
    for i in range(1, len(table)):
        v = jnp.where(idx == i, jnp.int32(table[i]), v)
    return v


def kernel(x):
    m, n = x.shape

    S = int(_os.environ.get("KERNEL_SUBCHUNKS", "2"))
    ms = m // S

    def body(x_ref, out_ref, send_x, recv_x, send_r, recv_r, send_l, recv_l,
             send_fr, recv_fr, send_fl, recv_fl):
        me = lax.axis_index("i")
        z = me // 8
        q = me % 8
        y = q // 2
        xs = jnp.where(y % 2 == 0, q % 2, 1 - (q % 2))
        p_me = 4 * z + y
        t = _tab(p_me, INV16)
        partner = jnp.bitwise_xor(me, 1)

        def ring_dev(tt):
            yy = _tab(tt, Y16)
            zz = _tab(tt, Z16)
            return 8 * zz + 2 * yy + jnp.where(yy % 2 == 0, xs, 1 - xs)

        right_id = ring_dev((t + 1) % N_PAIR)
        left_id = ring_dev((t + N_PAIR - 1) % N_PAIR)

        barrier = pltpu.get_barrier_semaphore()
        for nbr in (partner, left_id, right_id):
            pl.semaphore_signal(
                barrier, inc=1,
                device_id=(nbr,), device_id_type=pl.DeviceIdType.MESH,
            )
        pl.semaphore_wait(barrier, 3)

        out_ref[pl.ds(me * m, m), :] = x_ref[:, :].astype(out_ref.dtype)

        def desc(chunk, s, send_sems, recv_sems, idx, dev):
            row = chunk * m + s * ms
            return pltpu.make_async_remote_copy(
                src_ref=out_ref.at[pl.ds(row, ms)],
                dst_ref=out_ref.at[pl.ds(row, ms)],
                send_sem=send_sems.at[idx],
                recv_sem=recv_sems.at[idx],
                device_id=(dev,),
                device_id_type=pl.DeviceIdType.MESH,
            )

        def my_lane(pair):
            return 2 * pair + xs

        po_r = [_tab((t + N_PAIR - h) % N_PAIR, P16) for h in range(R_HOPS)]
        po_l = [_tab((t + h) % N_PAIR, P16) for h in range(L_HOPS)]
        po_in_r = [_tab((t + N_PAIR - 1 - k) % N_PAIR, P16)
                   for k in range(R_HOPS)]
        po_in_l = [_tab((t + 1 + k) % N_PAIR, P16) for k in range(L_HOPS)]

        rd_x = []
        for s in range(S):
            rd = desc(me, s, send_x, recv_x, s, partner)
            rd.start()
            rd_x.append(rd)

        inj = my_lane(p_me)
        for s in range(S):
            @pl.when(inj != me)
            def _():
                rd_x[s].wait_recv()
            desc(inj, s, send_r, recv_r, s, right_id).start()
            desc(inj, s, send_l, recv_l, s, left_id).start()

        rds_r = [[None] * S for _ in range(R_HOPS)]
        rds_l = [[None] * S for _ in range(L_HOPS)]
        fwds = []
        for h in range(1, R_HOPS):
            for s in range(S):
                prev = rds_r[h - 1][s]
                if prev is None:
                    prev = desc(my_lane(po_in_r[0]), s, send_r, recv_r,
                                s, right_id)
                prev.wait_recv()
                rd = desc(my_lane(po_r[h]), s, send_r, recv_r,
                          h * S + s, right_id)
                rd.start()
                rds_r[h][s] = rd
                fw = desc(my_lane(po_in_r[h - 1]), s, send_fr, recv_fr,
                          (h - 1) * S + s, partner)
                fw.start()
                fwds.append(fw)
                prev.wait_send()
                if h < L_HOPS:
                    prev = rds_l[h - 1][s]
                    if prev is None:
                        prev = desc(my_lane(po_in_l[0]), s, send_l, recv_l,
                                    s, left_id)
                    prev.wait_recv()
                    rd = desc(my_lane(po_l[h]), s, send_l, recv_l,
                              h * S + s, left_id)
                    rd.start()
                    rds_l[h][s] = rd
                    fw = desc(my_lane(po_in_l[h - 1]), s, send_fl, recv_fl,
                              (h - 1) * S + s, partner)
                    fw.start()
                    fwds.append(fw)
                    prev.wait_send()

        for s in range(S):
            rds_r[R_HOPS - 1][s].wait_recv()
            fw = desc(my_lane(po_in_r[R_HOPS - 1]), s, send_fr, recv_fr,
                      (R_HOPS - 1) * S + s, partner)
            fw.start()
            fwds.append(fw)
            rds_r[R_HOPS - 1][s].wait_send()
            rds_l[L_HOPS - 1][s].wait_recv()
            fw = desc(my_lane(po_in_l[L_HOPS - 1]), s, send_fl, recv_fl,
                      (L_HOPS - 1) * S + s, partner)
            fw.start()
            fwds.append(fw)
            rds_l[L_HOPS - 1][s].wait_send()

        other = 1 - xs
        for k in range(R_HOPS):
            for s in range(S):
                desc(2 * po_in_r[k] + other, s, send_fr, recv_fr,
                     k * S + s, partner).wait_recv()
        for k in range(L_HOPS):
            for s in range(S):
                desc(2 * po_in_l[k] + other, s, send_fl, recv_fl,
                     k * S + s, partner).wait_recv()
        for s in range(S):
            @pl.when(inj == me)
            def _():
                rd_x[s].wait_recv()
            rd_x[s].wait_send()
        for fw in fwds:
            fw.wait_send()

    return pl.pallas_call(
        body,
        out_shape=jax.ShapeDtypeStruct((N_DEV * m, n), jnp.bfloat16),
        in_specs=[pl.BlockSpec(memory_space=pltpu.VMEM)],
        out_specs=pl.BlockSpec(memory_space=pltpu.VMEM),
        scratch_shapes=[
            pltpu.SemaphoreType.DMA((S,)),
            pltpu.SemaphoreType.DMA((S,)),
            pltpu.SemaphoreType.DMA((R_HOPS * S,)),
            pltpu.SemaphoreType.DMA((R_HOPS * S,)),
            pltpu.SemaphoreType.DMA((L_HOPS * S,)),
            pltpu.SemaphoreType.DMA((L_HOPS * S,)),
            pltpu.SemaphoreType.DMA((R_HOPS * S,)),
            pltpu.SemaphoreType.DMA((R_HOPS * S,)),
            pltpu.SemaphoreType.DMA((L_HOPS * S,)),
            pltpu.SemaphoreType.DMA((L_HOPS * S,)),
        ],
        compiler_params=pltpu.CompilerParams(collective_id=0),
    )(x)


# device time: 33694 ns/iter; 1.0418x vs baseline; 1.0418x over previous
import os as _os

import jax
import jax.numpy as jnp
from jax import lax
from jax.experimental import pallas as pl
from jax.experimental.pallas import tpu as pltpu

N_DEV = 32
N_PAIR = 16
R_HOPS = 8
L_HOPS = 7

_CY = [(0, 0), (1, 0), (2, 0), (3, 0), (3, 1), (3, 2), (3, 3), (2, 3),
       (2, 2), (2, 1), (1, 1), (1, 2), (1, 3), (0, 3), (0, 2), (0, 1)]
Y16 = [y for y, _ in _CY]
Z16 = [z for _, z in _CY]
P16 = [4 * z + y for y, z in _CY]
INV16 = [0] * N_PAIR
for _t, _p in enumerate(P16):
    INV16[_p] = _t


def _tab(idx, table):
    v = jnp.int32(table[0])
    for i in range(1, len(table)):
        v = jnp.where(idx == i, jnp.int32(table[i]), v)
    return v


def kernel(x):
    m, n = x.shape

    S = int(_os.environ.get("KERNEL_SUBCHUNKS", "2"))
    ms = m // S

    def body(x_ref, out_ref, send_x, recv_x, send_r, recv_r, send_l, recv_l):
        me = lax.axis_index("i")
        z = me // 8
        q = me % 8
        y = q // 2
        xs = jnp.where(y % 2 == 0, q % 2, 1 - (q % 2))
        p_me = 4 * z + y
        t = _tab(p_me, INV16)
        my_par = me % 2
        partner = jnp.bitwise_xor(me, 1)

        def ring_dev(tt):
            yy = _tab(tt, Y16)
            zz = _tab(tt, Z16)
            return 8 * zz + 2 * yy + jnp.where(yy % 2 == 0, xs, 1 - xs)

        right_id = ring_dev((t + 1) % N_PAIR)
        left_id = ring_dev((t + N_PAIR - 1) % N_PAIR)

        barrier = pltpu.get_barrier_semaphore()
        for nbr in (partner, left_id, right_id):
            pl.semaphore_signal(
                barrier, inc=1,
                device_id=(nbr,), device_id_type=pl.DeviceIdType.MESH,
            )
        pl.semaphore_wait(barrier, 3)

        out_ref[pl.ds(me * m, m), :] = x_ref[:, :].astype(out_ref.dtype)

        def desc(row, send_sems, recv_sems, idx, dev):
            return pltpu.make_async_remote_copy(
                src_ref=out_ref.at[pl.ds(row, ms)],
                dst_ref=out_ref.at[pl.ds(row, ms)],
                send_sem=send_sems.at[idx],
                recv_sem=recv_sems.at[idx],
                device_id=(dev,),
                device_id_type=pl.DeviceIdType.MESH,
            )

        def idx(h, l, s):
            return h * 2 * S + l * S + s

        po_r = [_tab((t + N_PAIR - h) % N_PAIR, P16) for h in range(R_HOPS)]
        po_l = [_tab((t + h) % N_PAIR, P16) for h in range(L_HOPS)]
        po_in_r0 = _tab((t + N_PAIR - 1) % N_PAIR, P16)
        po_in_l0 = _tab((t + 1) % N_PAIR, P16)

        rd_x = []
        for s in range(S):
            rd = desc(me * m + s * ms, send_x, recv_x, s, partner)
            rd.start()
            rd_x.append(rd)
            own_idx = idx(0, my_par, s)
            desc(me * m + s * ms, send_r, recv_r, own_idx, right_id).start()
            desc(me * m + s * ms, send_l, recv_l, own_idx, left_id).start()
        for s in range(S):
            rd_x[s].wait_recv()
            p_idx = idx(0, 1 - my_par, s)
            row = partner * m + s * ms
            desc(row, send_r, recv_r, p_idx, right_id).start()
            desc(row, send_l, recv_l, p_idx, left_id).start()

        rds_r = [[[None] * S for _ in range(2)] for _ in range(R_HOPS)]
        rds_l = [[[None] * S for _ in range(2)] for _ in range(L_HOPS)]
        for h in range(1, R_HOPS):
            for l in (0, 1):
                for s in range(S):
                    prev = rds_r[h - 1][l][s]
                    if prev is None:
                        prev = desc((2 * po_in_r0 + l) * m + s * ms,
                                    send_r, recv_r, idx(0, l, s), right_id)
                    prev.wait_recv()
                    rd = desc((2 * po_r[h] + l) * m + s * ms,
                              send_r, recv_r, idx(h, l, s), right_id)
                    rd.start()
                    rds_r[h][l][s] = rd
                    prev.wait_send()
                    if h < L_HOPS:
                        prev = rds_l[h - 1][l][s]
                        if prev is None:
                            prev = desc((2 * po_in_l0 + l) * m + s * ms,
                                        send_l, recv_l, idx(0, l, s), left_id)
                        prev.wait_recv()
                        rd = desc((2 * po_l[h] + l) * m + s * ms,
                                  send_l, recv_l, idx(h, l, s), left_id)
                        rd.start()
                        rds_l[h][l][s] = rd
                        prev.wait_send()

        for l in (0, 1):
            for s in range(S):
                rds_r[R_HOPS - 1][l][s].wait_recv()
                rds_r[R_HOPS - 1][l][s].wait_send()
                rds_l[L_HOPS - 1][l][s].wait_recv()
                rds_l[L_HOPS - 1][l][s].wait_send()
        for s in range(S):
            rd_x[s].wait_send()

    return pl.pallas_call(
        body,
        out_shape=jax.ShapeDtypeStruct((N_DEV * m, n), jnp.bfloat16),
        in_specs=[pl.BlockSpec(memory_space=pltpu.VMEM)],
        out_specs=pl.BlockSpec(memory_space=pltpu.VMEM),
        scratch_shapes=[
            pltpu.SemaphoreType.DMA((S,)),
            pltpu.SemaphoreType.DMA((S,)),
            pltpu.SemaphoreType.DMA((R_HOPS * 2 * S,)),
            pltpu.SemaphoreType.DMA((R_HOPS * 2 * S,)),
            pltpu.SemaphoreType.DMA((L_HOPS * 2 * S,)),
            pltpu.SemaphoreType.DMA((L_HOPS * 2 * S,)),
        ],
        compiler_params=pltpu.CompilerParams(collective_id=0),
    )(x)
